# Initial kernel scaffold; baseline (speedup 1.0000x reference)
import jax
import jax.numpy as jnp
from jax.experimental import pallas as pl


def kernel(x, noise, Wg, bg, Wn, bn, W1, b1, W2, b2, W3, b3, Wl, bl, gamma, beta):
    raise NotImplementedError("write your pallas kernel here")


# speedup vs baseline: 4.1823x; 4.1823x over previous
import functools

import jax
import jax.numpy as jnp
from jax.experimental import pallas as pl
from jax.experimental.pallas import tpu as pltpu

_TOPK = 8


def _softplus(v):
    return jnp.maximum(v, 0.0) + jnp.log(1.0 + jnp.exp(-jnp.abs(v)))


def _gelu_exact(v):
    return 0.5 * v * (1.0 + jax.lax.erf(v * 0.7071067811865476))


def _router_body(x_ref, wg_ref, bg_ref, wn_ref, bn_ref, noise_ref, gates_ref):
    xv = x_ref[...]
    logits = jnp.dot(xv, wg_ref[...], preferred_element_type=jnp.float32)
    logits = logits + bg_ref[...]
    nl = jnp.dot(xv, wn_ref[...], preferred_element_type=jnp.float32)
    nl = nl + bn_ref[...]
    noisy = logits + noise_ref[...] * _softplus(nl)
    ncols = noisy.shape[-1]
    colid = jax.lax.broadcasted_iota(jnp.int32, noisy.shape, 1)
    work = noisy
    sel = jnp.zeros(noisy.shape, dtype=jnp.bool_)
    for _ in range(_TOPK):
        m = jnp.max(work, axis=-1, keepdims=True)
        ismax = work == m
        first = jnp.min(jnp.where(ismax, colid, ncols), axis=-1, keepdims=True)
        pick = colid == first
        sel = jnp.logical_or(sel, pick)
        work = jnp.where(pick, -jnp.inf, work)
    mx = jnp.max(jnp.where(sel, noisy, -jnp.inf), axis=-1, keepdims=True)
    ex = jnp.where(sel, jnp.exp(noisy - mx), 0.0)
    gates_ref[...] = ex / jnp.sum(ex, axis=-1, keepdims=True)


def _experts_body(x_ref, gates_ref, w1_ref, b1_ref, w2_ref, b2_ref, w3_ref,
                  b3_ref, wl_ref, bl_ref, gamma_ref, beta_ref, out_ref,
                  acc_ref):
    e = pl.program_id(0)
    ne = pl.num_programs(0)
    xv = x_ref[...]
    h = jnp.dot(xv, w1_ref[0], preferred_element_type=jnp.float32)
    h = _gelu_exact(h + b1_ref[0])
    h = jnp.dot(h, w2_ref[0], preferred_element_type=jnp.float32)
    h = _gelu_exact(h + b2_ref[0])
    h = jnp.dot(h, w3_ref[0], preferred_element_type=jnp.float32)
    h = h + b3_ref[0]
    eo = jnp.dot(h, wl_ref[0], preferred_element_type=jnp.float32)
    eo = eo + bl_ref[0]
    gates = gates_ref[...]
    colid = jax.lax.broadcasted_iota(jnp.int32, gates.shape, 1)
    gcol = jnp.sum(jnp.where(colid == e, gates, 0.0), axis=-1, keepdims=True)
    contrib = eo * gcol

    @pl.when(e == 0)
    def _():
        acc_ref[...] = contrib

    @pl.when(e > 0)
    def _():
        acc_ref[...] = acc_ref[...] + contrib

    @pl.when(e == ne - 1)
    def _():
        v = acc_ref[...]
        mu = jnp.mean(v, axis=-1, keepdims=True)
        c = v - mu
        var = jnp.mean(c * c, axis=-1, keepdims=True)
        out_ref[...] = c * jax.lax.rsqrt(var + 1e-5) * gamma_ref[...] \
            + beta_ref[...]


def kernel(x, noise, Wg, bg, Wn, bn, W1, b1, W2, b2, W3, b3, Wl, bl, gamma,
           beta):
    B, N, D = x.shape
    E = Wg.shape[1]
    H = W1.shape[2]
    K2 = W2.shape[2]
    BOT = W3.shape[2]
    OUT = Wl.shape[2]
    T = B * N
    x2 = x.reshape(T, D)
    noise2 = noise.reshape(T, E)

    gates = pl.pallas_call(
        _router_body,
        out_shape=jax.ShapeDtypeStruct((T, E), jnp.float32),
    )(x2, Wg, bg.reshape(1, E), Wn, bn.reshape(1, E), noise2)

    out = pl.pallas_call(
        _experts_body,
        grid=(E,),
        in_specs=[
            pl.BlockSpec((T, D), lambda e: (0, 0)),
            pl.BlockSpec((T, E), lambda e: (0, 0)),
            pl.BlockSpec((1, D, H), lambda e: (e, 0, 0)),
            pl.BlockSpec((1, 1, H), lambda e: (e, 0, 0)),
            pl.BlockSpec((1, H, K2), lambda e: (e, 0, 0)),
            pl.BlockSpec((1, 1, K2), lambda e: (e, 0, 0)),
            pl.BlockSpec((1, K2, BOT), lambda e: (e, 0, 0)),
            pl.BlockSpec((1, 1, BOT), lambda e: (e, 0, 0)),
            pl.BlockSpec((1, BOT, OUT), lambda e: (e, 0, 0)),
            pl.BlockSpec((1, 1, OUT), lambda e: (e, 0, 0)),
            pl.BlockSpec((1, OUT), lambda e: (0, 0)),
            pl.BlockSpec((1, OUT), lambda e: (0, 0)),
        ],
        out_specs=pl.BlockSpec((T, OUT), lambda e: (0, 0)),
        out_shape=jax.ShapeDtypeStruct((T, OUT), jnp.float32),
        scratch_shapes=[pltpu.VMEM((T, OUT), jnp.float32)],
        compiler_params=pltpu.CompilerParams(
            dimension_semantics=("arbitrary",)),
    )(x2, gates, W1, b1.reshape(E, 1, H), W2, b2.reshape(E, 1, K2),
      W3, b3.reshape(E, 1, BOT), Wl, bl.reshape(E, 1, OUT),
      gamma.reshape(1, OUT), beta.reshape(1, OUT))

    return out.reshape(B, N, OUT)


# speedup vs baseline: 5.3888x; 1.2885x over previous
import functools

import jax
import jax.numpy as jnp
from jax import lax
from jax.experimental import pallas as pl
from jax.experimental.pallas import tpu as pltpu
from jax.experimental.pallas import tpu_sc as plsc

_TOPK = 8
_BLK = 128
_BT = 128
_NC = 2
_NS = 16
_NW = _NC * _NS


def _softplus(v):
    return jnp.maximum(v, 0.0) + jnp.log(1.0 + jnp.exp(-jnp.abs(v)))


def _gelu_exact(v):
    return 0.5 * v * (1.0 + jax.lax.erf(v * 0.7071067811865476))


def _router_body(x_ref, noise_ref, wg_ref, bg_ref, wn_ref, bn_ref,
                 pos8_ref, gate8_ref, counts_ref, cnt_ref, off_ref,
                 call_ref):
    p = pl.program_id(0)
    b = pl.program_id(1)
    nb = pl.num_programs(1)

    @pl.when(jnp.logical_and(p == 0, b == 0))
    def _():
        cnt_ref[...] = jnp.zeros_like(cnt_ref)
        off_ref[...] = jnp.zeros_like(off_ref)

    xv = x_ref[...]
    logits = jnp.dot(xv, wg_ref[...], preferred_element_type=jnp.float32)
    logits = logits + bg_ref[...]
    nl = jnp.dot(xv, wn_ref[...], preferred_element_type=jnp.float32)
    nl = nl + bn_ref[...]
    noisy = logits + noise_ref[...] * _softplus(nl)

    ncols = noisy.shape[-1]
    colid = jax.lax.broadcasted_iota(jnp.int32, noisy.shape, 1)

    @pl.when(jnp.logical_and(p == 1, b == 0))
    def _():
        cntv = cnt_ref[0:1, :]
        upper = (jax.lax.broadcasted_iota(jnp.int32, (ncols, ncols), 0) <
                 jax.lax.broadcasted_iota(jnp.int32, (ncols, ncols), 1)
                 ).astype(jnp.float32)
        off_ref[0:1, :] = jnp.dot(cntv, upper,
                                  precision=jax.lax.Precision.HIGHEST,
                                  preferred_element_type=jnp.float32)

    posd = call_ref[pl.ds(b * _BT, _BT), :] + off_ref[0:1, :]

    work = noisy
    sel = jnp.zeros(noisy.shape, dtype=jnp.bool_)
    vals = []
    poss = []
    for _ in range(_TOPK):
        m = jnp.max(work, axis=-1, keepdims=True)
        ismax = work == m
        first = jnp.min(jnp.where(ismax, colid, ncols), axis=-1,
                        keepdims=True)
        pick = colid == first
        vals.append(m)
        poss.append(jnp.sum(jnp.where(pick, posd, 0.0), axis=-1,
                            keepdims=True))
        sel = jnp.logical_or(sel, pick)
        work = jnp.where(pick, -jnp.inf, work)

    mx = vals[0]
    exps = [jnp.exp(v - mx) for v in vals]
    denom = exps[0]
    for v in exps[1:]:
        denom = denom + v
    gate8 = jnp.concatenate([v / denom for v in exps], axis=1)
    pos8 = jnp.concatenate(poss, axis=1)

    pos8_ref[...] = pos8.astype(jnp.int32)
    gate8_ref[...] = gate8

    @pl.when(p == 0)
    def _():
        mask_f = sel.astype(jnp.float32)
        nrows = mask_f.shape[0]
        tril = (jax.lax.broadcasted_iota(jnp.int32, (nrows, nrows), 0) >
                jax.lax.broadcasted_iota(jnp.int32, (nrows, nrows), 1)
                ).astype(jnp.float32)
        call_ref[pl.ds(b * _BT, _BT), :] = (
            jnp.dot(tril, mask_f, precision=jax.lax.Precision.HIGHEST,
                    preferred_element_type=jnp.float32)
            + cnt_ref[0:1, :])
        newcnt = cnt_ref[0:1, :] + jnp.sum(mask_f, axis=0, keepdims=True)
        cnt_ref[0:1, :] = newcnt

        @pl.when(b == nb - 1)
        def _():
            counts_ref[...] = newcnt.astype(jnp.int32)


def _run_router(x2, noise2, Wg, bg, Wn, bn):
    T, D = x2.shape
    E = Wg.shape[1]
    nb = T // _BT
    return pl.pallas_call(
        _router_body,
        grid=(2, nb),
        in_specs=[
            pl.BlockSpec((_BT, D), lambda p, b: (b, 0)),
            pl.BlockSpec((_BT, E), lambda p, b: (b, 0)),
            pl.BlockSpec((D, E), lambda p, b: (0, 0)),
            pl.BlockSpec((1, E), lambda p, b: (0, 0)),
            pl.BlockSpec((D, E), lambda p, b: (0, 0)),
            pl.BlockSpec((1, E), lambda p, b: (0, 0)),
        ],
        out_specs=[
            pl.BlockSpec((_BT, _TOPK), lambda p, b: (b * p, 0)),
            pl.BlockSpec((_BT, _TOPK), lambda p, b: (b * p, 0)),
            pl.BlockSpec((1, E), lambda p, b: (0, 0)),
        ],
        out_shape=[
            jax.ShapeDtypeStruct((T, _TOPK), jnp.int32),
            jax.ShapeDtypeStruct((T, _TOPK), jnp.float32),
            jax.ShapeDtypeStruct((1, E), jnp.int32),
        ],
        scratch_shapes=[
            pltpu.VMEM((8, E), jnp.float32),
            pltpu.VMEM((8, E), jnp.float32),
            pltpu.VMEM((T, E), jnp.float32),
        ],
        compiler_params=pltpu.CompilerParams(
            dimension_semantics=("arbitrary", "arbitrary")),
    )(x2, noise2, Wg, bg.reshape(1, E), Wn, bn.reshape(1, E))


def _sc_dispatch(x2, posw):
    T, D = x2.shape
    A = T * _TOPK
    tw = T // _NW
    mesh = plsc.VectorSubcoreMesh(core_axis_name="c", subcore_axis_name="s")

    @functools.partial(
        pl.kernel,
        out_type=jax.ShapeDtypeStruct((A, D), jnp.float32),
        mesh=mesh,
        scratch_types=[
            pltpu.VMEM((_TOPK, tw), jnp.int32),
            pltpu.VMEM((tw, D), jnp.float32),
            pltpu.SemaphoreType.DMA,
        ],
    )
    def k(x_hbm, posw_hbm, xs_hbm, pos_v, rows_v, sem):
        wid = lax.axis_index("s") * _NC + lax.axis_index("c")
        base = wid * tw
        pltpu.sync_copy(posw_hbm.at[wid], pos_v)
        pltpu.sync_copy(x_hbm.at[pl.ds(base, tw)], rows_v)
        handles = [
            pltpu.async_copy(rows_v, xs_hbm.at[pos_v.at[kk]], sem)
            for kk in range(_TOPK)
        ]
        for h in handles:
            h.wait()

    return k(x2, posw)


def _gmm_body(em_ref, pm_ref, rs_ref, re_ref, xs_ref, w1_ref, b1_ref,
              w2_ref, b2_ref, w3_ref, b3_ref, wl_ref, bl_ref, eos_ref):
    i = pl.program_id(0)
    rs = rs_ref[i]
    re = re_ref[i]

    @pl.when(rs < re)
    def _():
        rowid = pm_ref[i] * _BLK + jax.lax.broadcasted_iota(
            jnp.int32, (_BLK, 1), 0)
        valid = jnp.logical_and(rowid >= rs, rowid < re)
        h = jnp.dot(xs_ref[...], w1_ref[0],
                    preferred_element_type=jnp.float32)
        h = _gelu_exact(h + b1_ref[0])
        h = jnp.dot(h, w2_ref[0], preferred_element_type=jnp.float32)
        h = _gelu_exact(h + b2_ref[0])
        h = jnp.dot(h, w3_ref[0], preferred_element_type=jnp.float32)
        h = h + b3_ref[0]
        eo = jnp.dot(h, wl_ref[0], preferred_element_type=jnp.float32)
        eo = eo + bl_ref[0]
        eos_ref[...] = jnp.where(valid, eo, eos_ref[...])


def _run_gmm(em, pm, rs, re, xs, W1, b1, W2, b2, W3, b3, Wl, bl, nlog):
    A, D = xs.shape
    E, _, H = W1.shape
    K2 = W2.shape[2]
    BOT = W3.shape[2]
    OUT = Wl.shape[2]
    grid_spec = pltpu.PrefetchScalarGridSpec(
        num_scalar_prefetch=4,
        grid=(nlog,),
        in_specs=[
            pl.BlockSpec((_BLK, D), lambda i, em, pm, rs, re: (pm[i], 0)),
            pl.BlockSpec((1, D, H), lambda i, em, pm, rs, re: (em[i], 0, 0)),
            pl.BlockSpec((1, 1, H), lambda i, em, pm, rs, re: (em[i], 0, 0)),
            pl.BlockSpec((1, H, K2), lambda i, em, pm, rs, re: (em[i], 0, 0)),
            pl.BlockSpec((1, 1, K2), lambda i, em, pm, rs, re: (em[i], 0, 0)),
            pl.BlockSpec((1, K2, BOT),
                         lambda i, em, pm, rs, re: (em[i], 0, 0)),
            pl.BlockSpec((1, 1, BOT),
                         lambda i, em, pm, rs, re: (em[i], 0, 0)),
            pl.BlockSpec((1, BOT, OUT),
                         lambda i, em, pm, rs, re: (em[i], 0, 0)),
            pl.BlockSpec((1, 1, OUT),
                         lambda i, em, pm, rs, re: (em[i], 0, 0)),
        ],
        out_specs=pl.BlockSpec((_BLK, OUT),
                               lambda i, em, pm, rs, re: (pm[i], 0)),
    )
    return pl.pallas_call(
        _gmm_body,
        grid_spec=grid_spec,
        out_shape=jax.ShapeDtypeStruct((A, OUT), jnp.float32),
        compiler_params=pltpu.CompilerParams(
            dimension_semantics=("arbitrary",)),
    )(em, pm, rs, re, xs, W1, b1.reshape(E, 1, H), W2,
      b2.reshape(E, 1, K2), W3, b3.reshape(E, 1, BOT), Wl,
      bl.reshape(E, 1, OUT))


def _sc_combine(eos, posw):
    A, D = eos.shape
    T = A // _TOPK
    tw = T // _NW
    mesh = plsc.VectorSubcoreMesh(core_axis_name="c", subcore_axis_name="s")

    @functools.partial(
        pl.kernel,
        out_type=jax.ShapeDtypeStruct((_TOPK, T, D), jnp.float32),
        mesh=mesh,
        scratch_types=[
            pltpu.VMEM((_TOPK, tw), jnp.int32),
            pltpu.VMEM((tw, D), jnp.float32),
            pltpu.SemaphoreType.DMA,
        ],
    )
    def k(eos_hbm, posw_hbm, eok_hbm, pos_v, buf_v, sem):
        wid = lax.axis_index("s") * _NC + lax.axis_index("c")
        base = wid * tw
        pltpu.sync_copy(posw_hbm.at[wid], pos_v)
        for kk in range(_TOPK):
            pltpu.async_copy(eos_hbm.at[pos_v.at[kk]], buf_v, sem).wait()
            pltpu.sync_copy(buf_v, eok_hbm.at[kk, pl.ds(base, tw)])

    return k(eos, posw)


def _combine_body(eok_ref, g8_ref, gamma_ref, beta_ref, out_ref):
    g8 = g8_ref[...]
    acc = eok_ref[0] * g8[:, 0:1]
    for kk in range(1, _TOPK):
        acc = acc + eok_ref[kk] * g8[:, kk:kk + 1]
    mu = jnp.mean(acc, axis=-1, keepdims=True)
    c = acc - mu
    var = jnp.mean(c * c, axis=-1, keepdims=True)
    out_ref[...] = c * jax.lax.rsqrt(var + 1e-5) * gamma_ref[...] \
        + beta_ref[...]


def _run_combine(eok, gate8, gamma, beta):
    _, T, D = eok.shape
    nb = T // _BT
    return pl.pallas_call(
        _combine_body,
        grid=(nb,),
        in_specs=[
            pl.BlockSpec((_TOPK, _BT, D), lambda b: (0, b, 0)),
            pl.BlockSpec((_BT, _TOPK), lambda b: (b, 0)),
            pl.BlockSpec((1, D), lambda b: (0, 0)),
            pl.BlockSpec((1, D), lambda b: (0, 0)),
        ],
        out_specs=pl.BlockSpec((_BT, D), lambda b: (b, 0)),
        out_shape=jax.ShapeDtypeStruct((T, D), jnp.float32),
        compiler_params=pltpu.CompilerParams(
            dimension_semantics=("arbitrary",)),
    )(eok, gate8, gamma.reshape(1, D), beta.reshape(1, D))


def _block_metadata(counts, A, E):
    np_ = A // _BLK
    nlog = np_ + E - 1
    ends = jnp.cumsum(counts)
    starts = ends - counts
    first = starts // _BLK
    last = jnp.where(counts > 0, (ends - 1) // _BLK, first)
    nb = jnp.where(counts > 0, last - first + 1, 0)
    cnb = jnp.cumsum(nb)
    j = jnp.arange(nlog, dtype=jnp.int32)
    e_j = jnp.searchsorted(cnb, j, side="right").astype(jnp.int32)
    valid = e_j < E
    e_c = jnp.minimum(e_j, E - 1)
    cnb_ex = jnp.concatenate([jnp.zeros((1,), cnb.dtype), cnb[:-1]])
    local = j - cnb_ex[e_c]
    phys = jnp.where(valid, first[e_c] + local, np_ - 1).astype(jnp.int32)
    rs = jnp.where(valid, jnp.maximum(starts[e_c], phys * _BLK),
                   1).astype(jnp.int32)
    re = jnp.where(valid, jnp.minimum(ends[e_c], (phys + 1) * _BLK),
                   0).astype(jnp.int32)
    return e_c.astype(jnp.int32), phys, rs, re, nlog


def kernel(x, noise, Wg, bg, Wn, bn, W1, b1, W2, b2, W3, b3, Wl, bl, gamma,
           beta):
    B, N, D = x.shape
    E = Wg.shape[1]
    OUT = Wl.shape[2]
    T = B * N
    A = T * _TOPK
    x2 = x.reshape(T, D)
    noise2 = noise.reshape(T, E)

    pos8, gate8, counts2 = _run_router(x2, noise2, Wg, bg, Wn, bn)
    posw = pos8.T.reshape(_TOPK, _NW, T // _NW).transpose(1, 0, 2)

    xs = _sc_dispatch(x2, posw)
    em, pm, rs, re, nlog = _block_metadata(counts2[0], A, E)
    eos = _run_gmm(em, pm, rs, re, xs, W1, b1, W2, b2, W3, b3, Wl, bl, nlog)
    eok = _sc_combine(eos, posw)
    out = _run_combine(eok, gate8, gamma, beta)
    return out.reshape(B, N, OUT)
